# Initial kernel scaffold; baseline (speedup 1.0000x reference)
import jax
import jax.numpy as jnp
from jax.experimental import pallas as pl


def kernel(x, edge_index, batch, W1_0, b1_0, W2_0, b2_0, W1_1, b1_1, W2_1, b2_1):
    raise NotImplementedError("write your pallas kernel here")


# speedup vs baseline: 5.0866x; 5.0866x over previous
import functools

import jax
import jax.numpy as jnp
from jax import lax
from jax.experimental import pallas as pl
from jax.experimental.pallas import tpu as pltpu
from jax.experimental.pallas import tpu_sc as plsc

NN = 10000
NPAD = 10240
EE = 320000
DD = 128
BB = 64
NTILES = 32
E_PER_TILE = EE // NTILES
CH = 80
NCHUNK = E_PER_TILE // CH
ROWS_PER_TILE = NPAD // 16
RB = 400
NRB = NN // RB


def _sc_agg(table, src, dst):
    mesh = plsc.VectorSubcoreMesh(core_axis_name="c", subcore_axis_name="s")

    @functools.partial(
        pl.kernel,
        out_type=jax.ShapeDtypeStruct((2 * NPAD, DD), jnp.float32),
        mesh=mesh,
        scratch_types=[
            pltpu.VMEM((CH,), jnp.int32),
            pltpu.VMEM((CH,), jnp.int32),
            pltpu.VMEM((CH, DD), jnp.float32),
            pltpu.VMEM_SHARED((NPAD, DD), jnp.float32),
            pltpu.SemaphoreType.DMA,
        ],
    )
    def k(table_h, src_h, dst_h, out_h, src_v, dst_v, rows_v, acc_s, sem):
        cid = lax.axis_index("c")
        sid = lax.axis_index("s")
        tile = cid * 16 + sid

        def zrow(r, _):
            def zcol(j, _):
                rows_v[r, pl.ds(j * 16, 16)] = jnp.zeros((16,), jnp.float32)
                return 0
            return lax.fori_loop(0, DD // 16, zcol, 0)
        lax.fori_loop(0, CH, zrow, 0)

        def zacc(i, _):
            pltpu.sync_copy(
                rows_v, acc_s.at[pl.ds(sid * ROWS_PER_TILE + i * CH, CH)])
            return 0
        lax.fori_loop(0, ROWS_PER_TILE // CH, zacc, 0)
        plsc.subcore_barrier()

        base = tile * E_PER_TILE

        def body(j, _):
            off = pl.multiple_of(base + j * CH, 8)
            pltpu.sync_copy(src_h.at[pl.ds(off, CH)], src_v)
            pltpu.sync_copy(dst_h.at[pl.ds(off, CH)], dst_v)
            pltpu.async_copy(table_h.at[src_v], rows_v, sem).wait()
            pltpu.sync_copy(rows_v, acc_s.at[dst_v], add=True)
            return 0
        lax.fori_loop(0, NCHUNK, body, 0)
        plsc.subcore_barrier()

        pltpu.sync_copy(
            acc_s.at[pl.ds(sid * ROWS_PER_TILE, ROWS_PER_TILE)],
            out_h.at[pl.ds(cid * NPAD + sid * ROWS_PER_TILE, ROWS_PER_TILE)])

    return k(table, src, dst)


def _mlp0(x, a0, a1, W1, b1, W2, b2):
    def body(x_r, a0_r, a1_r, w1_r, b1_r, w2_r, b2_r, o_r):
        h = x_r[...] + a0_r[...] + a1_r[...]
        h = jnp.maximum(
            jnp.dot(h, w1_r[...], preferred_element_type=jnp.float32)
            + b1_r[...], 0.0)
        h = jnp.dot(h, w2_r[...], preferred_element_type=jnp.float32) + b2_r[...]
        o_r[...] = jnp.maximum(h, 0.0)

    row = pl.BlockSpec((RB, DD), lambda i: (i, 0))
    full = pl.BlockSpec((DD, DD), lambda i: (0, 0))
    bias = pl.BlockSpec((1, DD), lambda i: (0, 0))
    return pl.pallas_call(
        body,
        grid=(NRB,),
        in_specs=[row, row, row, full, bias, full, bias],
        out_specs=row,
        out_shape=jax.ShapeDtypeStruct((NN, DD), jnp.float32),
    )(x, a0, a1, W1, b1.reshape(1, DD), W2, b2.reshape(1, DD))


def _mlp1_pool(h0, a0, a1, W1, b1, W2, b2, batch3d):
    def body(h_r, a0_r, a1_r, w1_r, b1_r, w2_r, b2_r, bt_r, o_r, acc, cnt):
        i = pl.program_id(0)

        @pl.when(i == 0)
        def _():
            acc[...] = jnp.zeros_like(acc)
            cnt[...] = jnp.zeros_like(cnt)

        h = h_r[...] + a0_r[...] + a1_r[...]
        h = jnp.maximum(
            jnp.dot(h, w1_r[...], preferred_element_type=jnp.float32)
            + b1_r[...], 0.0)
        h = jnp.dot(h, w2_r[...], preferred_element_type=jnp.float32) + b2_r[...]

        seg = bt_r[...].reshape(1, RB)
        onehot = (jnp.broadcast_to(seg, (BB, RB))
                  == lax.broadcasted_iota(jnp.int32, (BB, RB), 0)
                  ).astype(jnp.float32)
        acc[...] += jnp.dot(onehot, h, preferred_element_type=jnp.float32)
        cnt[...] += jnp.broadcast_to(
            jnp.sum(onehot, axis=1, keepdims=True), (BB, DD))

        @pl.when(i == NRB - 1)
        def _():
            o_r[...] = acc[...] / jnp.maximum(cnt[...], 1.0)

    row = pl.BlockSpec((RB, DD), lambda i: (i, 0))
    full = pl.BlockSpec((DD, DD), lambda i: (0, 0))
    bias = pl.BlockSpec((1, DD), lambda i: (0, 0))
    return pl.pallas_call(
        body,
        grid=(NRB,),
        in_specs=[row, row, row, full, bias, full, bias,
                  pl.BlockSpec((1, 1, RB), lambda i: (i, 0, 0))],
        out_specs=pl.BlockSpec((BB, DD), lambda i: (0, 0)),
        out_shape=jax.ShapeDtypeStruct((BB, DD), jnp.float32),
        scratch_shapes=[pltpu.VMEM((BB, DD), jnp.float32),
                        pltpu.VMEM((BB, DD), jnp.float32)],
    )(h0, a0, a1, W1, b1.reshape(1, DD), W2, b2.reshape(1, DD), batch3d)


def kernel(x, edge_index, batch, W1_0, b1_0, W2_0, b2_0, W1_1, b1_1, W2_1, b2_1):
    src = edge_index[0]
    dst = edge_index[1]
    batch3d = batch.reshape(NRB, 1, RB)

    aggs = _sc_agg(x, src, dst)
    a0 = aggs[:NN]
    a1 = aggs[NPAD:NPAD + NN]
    h0 = _mlp0(x, a0, a1, W1_0, b1_0, W2_0, b2_0)

    aggs1 = _sc_agg(h0, src, dst)
    a0b = aggs1[:NN]
    a1b = aggs1[NPAD:NPAD + NN]
    return _mlp1_pool(h0, a0b, a1b, W1_1, b1_1, W2_1, b2_1, batch3d)


# speedup vs baseline: 11.0356x; 2.1695x over previous
import functools

import jax
import jax.numpy as jnp
from jax import lax
from jax.experimental import pallas as pl
from jax.experimental.pallas import tpu as pltpu
from jax.experimental.pallas import tpu_sc as plsc

NN = 10000
NPAD = 10240
EE = 320000
DD = 128
BB = 64
NTILES = 32
E_PER_TILE = EE // NTILES
CH = 80
NCHUNK = E_PER_TILE // CH
NBUF = 2
ROWS_PER_TILE = NPAD // 16
RB = 400
NRB = NN // RB


def _sc_agg(table, src2, dst3):
    mesh = plsc.VectorSubcoreMesh(core_axis_name="c", subcore_axis_name="s")

    @functools.partial(
        pl.kernel,
        out_type=jax.ShapeDtypeStruct((2 * NPAD, DD), jnp.float32),
        mesh=mesh,
        scratch_types=[
            pltpu.VMEM((E_PER_TILE,), jnp.int32),
            pltpu.VMEM((NCHUNK, CH), jnp.int32),
            [pltpu.VMEM((CH, DD), jnp.float32) for _ in range(NBUF)],
            pltpu.VMEM_SHARED((NPAD, DD), jnp.float32),
            [pltpu.SemaphoreType.DMA for _ in range(NBUF)],
        ],
    )
    def k(table_h, src_h, dst_h, out_h, src_v, dst_v, rows, acc_s, gsem):
        cid = lax.axis_index("c")
        sid = lax.axis_index("s")
        tile = cid * 16 + sid

        pltpu.sync_copy(src_h.at[tile], src_v)
        pltpu.sync_copy(dst_h.at[tile], dst_v)

        def zrow(r, _):
            def zcol(j, _):
                rows[0][r, pl.ds(j * 16, 16)] = jnp.zeros((16,), jnp.float32)
                return 0
            return lax.fori_loop(0, DD // 16, zcol, 0)
        lax.fori_loop(0, CH, zrow, 0)

        def zacc(i, _):
            pltpu.sync_copy(
                rows[0], acc_s.at[pl.ds(sid * ROWS_PER_TILE + i * CH, CH)])
            return 0
        lax.fori_loop(0, ROWS_PER_TILE // CH, zacc, 0)
        plsc.subcore_barrier()

        def gidx(j):
            return src_v.at[pl.ds(pl.multiple_of(j * CH, 8), CH)]

        for b in range(NBUF):
            pltpu.async_copy(table_h.at[gidx(b)], rows[b], gsem[b])

        def body(o, _):
            for b in range(NBUF):
                j = o * NBUF + b
                pltpu.make_async_copy(table_h.at[gidx(j)], rows[b],
                                      gsem[b]).wait()
                pltpu.sync_copy(rows[b], acc_s.at[dst_v.at[j]], add=True)
                pltpu.async_copy(table_h.at[gidx(j + NBUF)], rows[b],
                                 gsem[b])
            return 0
        lax.fori_loop(0, (NCHUNK - 1) // NBUF - 1, body, 0)
        for b in range(NBUF):
            j = NCHUNK - 1 - NBUF + b
            pltpu.make_async_copy(table_h.at[gidx(j)], rows[b],
                                  gsem[b]).wait()
            pltpu.sync_copy(rows[b], acc_s.at[dst_v.at[j]], add=True)
        j = NCHUNK - 1
        pltpu.async_copy(table_h.at[gidx(j)], rows[0], gsem[0])
        pltpu.make_async_copy(table_h.at[gidx(j)], rows[0], gsem[0]).wait()
        pltpu.sync_copy(rows[0], acc_s.at[dst_v.at[j]], add=True)
        plsc.subcore_barrier()

        pltpu.sync_copy(
            acc_s.at[pl.ds(sid * ROWS_PER_TILE, ROWS_PER_TILE)],
            out_h.at[pl.ds(cid * NPAD + sid * ROWS_PER_TILE, ROWS_PER_TILE)])

    return k(table, src2, dst3)


def _mlp0(x, a0, a1, W1, b1, W2, b2):
    def body(x_r, a0_r, a1_r, w1_r, b1_r, w2_r, b2_r, o_r):
        h = x_r[...] + a0_r[...] + a1_r[...]
        h = jnp.maximum(
            jnp.dot(h, w1_r[...], preferred_element_type=jnp.float32)
            + b1_r[...], 0.0)
        h = jnp.dot(h, w2_r[...], preferred_element_type=jnp.float32) + b2_r[...]
        o_r[...] = jnp.maximum(h, 0.0)

    row = pl.BlockSpec((RB, DD), lambda i: (i, 0))
    full = pl.BlockSpec((DD, DD), lambda i: (0, 0))
    bias = pl.BlockSpec((1, DD), lambda i: (0, 0))
    return pl.pallas_call(
        body,
        grid=(NRB,),
        in_specs=[row, row, row, full, bias, full, bias],
        out_specs=row,
        out_shape=jax.ShapeDtypeStruct((NN, DD), jnp.float32),
    )(x, a0, a1, W1, b1.reshape(1, DD), W2, b2.reshape(1, DD))


def _mlp1_pool(h0, a0, a1, W1, b1, W2, b2, batch3d):
    def body(h_r, a0_r, a1_r, w1_r, b1_r, w2_r, b2_r, bt_r, o_r, acc, cnt):
        i = pl.program_id(0)

        @pl.when(i == 0)
        def _():
            acc[...] = jnp.zeros_like(acc)
            cnt[...] = jnp.zeros_like(cnt)

        h = h_r[...] + a0_r[...] + a1_r[...]
        h = jnp.maximum(
            jnp.dot(h, w1_r[...], preferred_element_type=jnp.float32)
            + b1_r[...], 0.0)
        h = jnp.dot(h, w2_r[...], preferred_element_type=jnp.float32) + b2_r[...]

        seg = bt_r[...].reshape(1, RB)
        onehot = (jnp.broadcast_to(seg, (BB, RB))
                  == lax.broadcasted_iota(jnp.int32, (BB, RB), 0)
                  ).astype(jnp.float32)
        acc[...] += jnp.dot(onehot, h, preferred_element_type=jnp.float32)
        cnt[...] += jnp.broadcast_to(
            jnp.sum(onehot, axis=1, keepdims=True), (BB, DD))

        @pl.when(i == NRB - 1)
        def _():
            o_r[...] = acc[...] / jnp.maximum(cnt[...], 1.0)

    row = pl.BlockSpec((RB, DD), lambda i: (i, 0))
    full = pl.BlockSpec((DD, DD), lambda i: (0, 0))
    bias = pl.BlockSpec((1, DD), lambda i: (0, 0))
    return pl.pallas_call(
        body,
        grid=(NRB,),
        in_specs=[row, row, row, full, bias, full, bias,
                  pl.BlockSpec((1, 1, RB), lambda i: (i, 0, 0))],
        out_specs=pl.BlockSpec((BB, DD), lambda i: (0, 0)),
        out_shape=jax.ShapeDtypeStruct((BB, DD), jnp.float32),
        scratch_shapes=[pltpu.VMEM((BB, DD), jnp.float32),
                        pltpu.VMEM((BB, DD), jnp.float32)],
    )(h0, a0, a1, W1, b1.reshape(1, DD), W2, b2.reshape(1, DD), batch3d)


def kernel(x, edge_index, batch, W1_0, b1_0, W2_0, b2_0, W1_1, b1_1, W2_1, b2_1):
    src2 = edge_index[0].reshape(NTILES, E_PER_TILE)
    dst3 = edge_index[1].reshape(NTILES, NCHUNK, CH)
    batch3d = batch.reshape(NRB, 1, RB)

    aggs = _sc_agg(x, src2, dst3)
    a0 = aggs[:NN]
    a1 = aggs[NPAD:NPAD + NN]
    h0 = _mlp0(x, a0, a1, W1_0, b1_0, W2_0, b2_0)

    aggs1 = _sc_agg(h0, src2, dst3)
    a0b = aggs1[:NN]
    a1b = aggs1[NPAD:NPAD + NN]
    return _mlp1_pool(h0, a0b, a1b, W1_1, b1_1, W2_1, b2_1, batch3d)


# speedup vs baseline: 11.6737x; 1.0578x over previous
import functools

import jax
import jax.numpy as jnp
from jax import lax
from jax.experimental import pallas as pl
from jax.experimental.pallas import tpu as pltpu
from jax.experimental.pallas import tpu_sc as plsc

NN = 10000
NPAD = 10112
EE = 320000
DD = 128
BB = 64
NTILES = 32
E_PER_TILE = EE // NTILES
CH = 80
NCHUNK = E_PER_TILE // CH
NBUF = 4
NIDX = 8
ROWS_PER_TILE = NPAD // 16
ZCH = ROWS_PER_TILE // 8
RB = 400
NRB = NN // RB


def _sc_agg(table, ei4):
    mesh = plsc.VectorSubcoreMesh(core_axis_name="c", subcore_axis_name="s")

    @functools.partial(
        pl.kernel,
        out_type=jax.ShapeDtypeStruct((2 * NPAD, DD), jnp.float32),
        mesh=mesh,
        scratch_types=[
            pltpu.VMEM((NIDX, 2, CH), jnp.int32),
            [pltpu.VMEM((CH, DD), jnp.float32) for _ in range(NBUF)],
            pltpu.VMEM_SHARED((NPAD, DD), jnp.float32),
            [pltpu.SemaphoreType.DMA for _ in range(NIDX)],
            [pltpu.SemaphoreType.DMA for _ in range(NBUF)],
            [pltpu.SemaphoreType.DMA for _ in range(NBUF)],
        ],
    )
    def k(table_h, ei_h, out_h, ring, rows, acc_s, isem, gsem, ssem):
        cid = lax.axis_index("c")
        sid = lax.axis_index("s")
        tile = cid * 16 + sid

        def zrow(r, _):
            def zcol(j, _):
                rows[0][r, pl.ds(j * 16, 16)] = jnp.zeros((16,), jnp.float32)
                return 0
            return lax.fori_loop(0, DD // 16, zcol, 0)
        lax.fori_loop(0, CH, zrow, 0)

        def zacc(i, _):
            pltpu.sync_copy(
                rows[0].at[pl.ds(0, ZCH)],
                acc_s.at[pl.ds(sid * ROWS_PER_TILE + i * ZCH, ZCH)])
            return 0
        lax.fori_loop(0, ROWS_PER_TILE // ZCH, zacc, 0)
        plsc.subcore_barrier()

        def iload(c, s):
            pltpu.async_copy(ei_h.at[tile, c], ring.at[s], isem[s])

        def iwait(c, s):
            pltpu.make_async_copy(ei_h.at[tile, c], ring.at[s],
                                  isem[s]).wait()

        def gstart(s, b):
            pltpu.async_copy(table_h.at[ring.at[s, 0]], rows[b], gsem[b])

        def gwait(s, b):
            pltpu.make_async_copy(table_h.at[ring.at[s, 0]], rows[b],
                                  gsem[b]).wait()

        def sstart(s, b):
            pltpu.async_copy(rows[b], acc_s.at[ring.at[s, 1]], ssem[b],
                             add=True)

        def swait(s, b):
            pltpu.make_async_copy(rows[b], acc_s.at[ring.at[s, 1]],
                                  ssem[b]).wait()

        def iter_ops(j, jm8, jm4, do_swait, do_iload, do_gather):
            gwait(jm8, jm4)
            sstart(jm8, jm4)
            if do_swait:
                swait((jm8 + 2) % NIDX, (jm4 + 2) % NBUF)
            if do_iload:
                iload(j + 6, (jm8 + 6) % NIDX)
            if do_gather:
                iwait(j + 2, (jm8 + 2) % NIDX)
                gstart((jm8 + 2) % NIDX, (jm4 + 2) % NBUF)

        for c in range(6):
            iload(c, c)
        for c in range(2):
            iwait(c, c)
            gstart(c, c)
        for j in (0, 1):
            iter_ops(j, j % NIDX, j % NBUF, False, True, True)
        for j in range(2, 10):
            iter_ops(j, j % NIDX, j % NBUF, True, True, True)

        def body(o, _):
            j0 = 10 + o * 8
            for t in range(8):
                iter_ops(j0 + t, (10 + t) % NIDX, (10 + t) % NBUF,
                         True, True, True)
            return 0
        lax.fori_loop(0, 13, body, 0)

        for j in range(114, 119):
            iter_ops(j, j % NIDX, j % NBUF, True, True, True)
        for j in range(119, 123):
            iter_ops(j, j % NIDX, j % NBUF, True, False, True)
        for j in (123, 124):
            jm8, jm4 = j % NIDX, j % NBUF
            gwait(jm8, jm4)
            sstart(jm8, jm4)
            swait((jm8 + 2) % NIDX, (jm4 + 2) % NBUF)
        for j in (123, 124):
            swait(j % NIDX, j % NBUF)
        plsc.subcore_barrier()

        pltpu.sync_copy(
            acc_s.at[pl.ds(sid * ROWS_PER_TILE, ROWS_PER_TILE)],
            out_h.at[pl.ds(cid * NPAD + sid * ROWS_PER_TILE, ROWS_PER_TILE)])

    return k(table, ei4)


def _mlp0(x, a0, a1, W1, b1, W2, b2):
    def body(x_r, a0_r, a1_r, w1_r, b1_r, w2_r, b2_r, o_r):
        h = x_r[...] + a0_r[...] + a1_r[...]
        h = jnp.maximum(
            jnp.dot(h, w1_r[...], preferred_element_type=jnp.float32)
            + b1_r[...], 0.0)
        h = jnp.dot(h, w2_r[...], preferred_element_type=jnp.float32) + b2_r[...]
        o_r[...] = jnp.maximum(h, 0.0)

    row = pl.BlockSpec((RB, DD), lambda i: (i, 0))
    full = pl.BlockSpec((DD, DD), lambda i: (0, 0))
    bias = pl.BlockSpec((1, DD), lambda i: (0, 0))
    return pl.pallas_call(
        body,
        grid=(NRB,),
        in_specs=[row, row, row, full, bias, full, bias],
        out_specs=row,
        out_shape=jax.ShapeDtypeStruct((NN, DD), jnp.float32),
    )(x, a0, a1, W1, b1.reshape(1, DD), W2, b2.reshape(1, DD))


def _mlp1_pool(h0, a0, a1, W1, b1, W2, b2, batch3d):
    def body(h_r, a0_r, a1_r, w1_r, b1_r, w2_r, b2_r, bt_r, o_r, acc, cnt):
        i = pl.program_id(0)

        @pl.when(i == 0)
        def _():
            acc[...] = jnp.zeros_like(acc)
            cnt[...] = jnp.zeros_like(cnt)

        h = h_r[...] + a0_r[...] + a1_r[...]
        h = jnp.maximum(
            jnp.dot(h, w1_r[...], preferred_element_type=jnp.float32)
            + b1_r[...], 0.0)
        h = jnp.dot(h, w2_r[...], preferred_element_type=jnp.float32) + b2_r[...]

        seg = bt_r[...].reshape(1, RB)
        onehot = (jnp.broadcast_to(seg, (BB, RB))
                  == lax.broadcasted_iota(jnp.int32, (BB, RB), 0)
                  ).astype(jnp.float32)
        acc[...] += jnp.dot(onehot, h, preferred_element_type=jnp.float32)
        cnt[...] += jnp.broadcast_to(
            jnp.sum(onehot, axis=1, keepdims=True), (BB, DD))

        @pl.when(i == NRB - 1)
        def _():
            o_r[...] = acc[...] / jnp.maximum(cnt[...], 1.0)

    row = pl.BlockSpec((RB, DD), lambda i: (i, 0))
    full = pl.BlockSpec((DD, DD), lambda i: (0, 0))
    bias = pl.BlockSpec((1, DD), lambda i: (0, 0))
    return pl.pallas_call(
        body,
        grid=(NRB,),
        in_specs=[row, row, row, full, bias, full, bias,
                  pl.BlockSpec((1, 1, RB), lambda i: (i, 0, 0))],
        out_specs=pl.BlockSpec((BB, DD), lambda i: (0, 0)),
        out_shape=jax.ShapeDtypeStruct((BB, DD), jnp.float32),
        scratch_shapes=[pltpu.VMEM((BB, DD), jnp.float32),
                        pltpu.VMEM((BB, DD), jnp.float32)],
    )(h0, a0, a1, W1, b1.reshape(1, DD), W2, b2.reshape(1, DD), batch3d)


def kernel(x, edge_index, batch, W1_0, b1_0, W2_0, b2_0, W1_1, b1_1, W2_1, b2_1):
    ei4 = edge_index.reshape(2, NTILES, NCHUNK, CH).transpose(1, 2, 0, 3)
    batch3d = batch.reshape(NRB, 1, RB)

    aggs = _sc_agg(x, ei4)
    a0 = aggs[:NN]
    a1 = aggs[NPAD:NPAD + NN]
    h0 = _mlp0(x, a0, a1, W1_0, b1_0, W2_0, b2_0)

    aggs1 = _sc_agg(h0, ei4)
    a0b = aggs1[:NN]
    a1b = aggs1[NPAD:NPAD + NN]
    return _mlp1_pool(h0, a0b, a1b, W1_1, b1_1, W2_1, b2_1, batch3d)


# speedup vs baseline: 12.5452x; 1.0746x over previous
import functools

import jax
import jax.numpy as jnp
from jax import lax
from jax.experimental import pallas as pl
from jax.experimental.pallas import tpu as pltpu
from jax.experimental.pallas import tpu_sc as plsc

NN = 10000
NPAD = 10112
EE = 320000
DD = 128
BB = 64
NTILES = 32
E_PER_TILE = EE // NTILES
CH = 80
NCHUNK = E_PER_TILE // CH
NBUF = 4
NIDX = 8
ROWS_PER_TILE = NPAD // 16
ZCH = ROWS_PER_TILE // 8
RB = 632
NRB = NPAD // RB


def _sc_agg(table, ei4):
    mesh = plsc.VectorSubcoreMesh(core_axis_name="c", subcore_axis_name="s")

    @functools.partial(
        pl.kernel,
        out_type=jax.ShapeDtypeStruct((2 * NPAD, DD), jnp.float32),
        mesh=mesh,
        scratch_types=[
            pltpu.VMEM((NIDX, 2, CH), jnp.int32),
            [pltpu.VMEM((CH, DD), jnp.float32) for _ in range(NBUF)],
            pltpu.VMEM_SHARED((NPAD, DD), jnp.float32),
            [pltpu.SemaphoreType.DMA for _ in range(NIDX)],
            [pltpu.SemaphoreType.DMA for _ in range(NBUF)],
            [pltpu.SemaphoreType.DMA for _ in range(NBUF)],
        ],
    )
    def k(table_h, ei_h, out_h, ring, rows, acc_s, isem, gsem, ssem):
        cid = lax.axis_index("c")
        sid = lax.axis_index("s")
        tile = cid * 16 + sid

        def zrow(r, _):
            def zcol(j, _):
                rows[0][r, pl.ds(j * 16, 16)] = jnp.zeros((16,), jnp.float32)
                return 0
            return lax.fori_loop(0, DD // 16, zcol, 0)
        lax.fori_loop(0, CH, zrow, 0)

        def zacc(i, _):
            pltpu.sync_copy(
                rows[0].at[pl.ds(0, ZCH)],
                acc_s.at[pl.ds(sid * ROWS_PER_TILE + i * ZCH, ZCH)])
            return 0
        lax.fori_loop(0, ROWS_PER_TILE // ZCH, zacc, 0)
        plsc.subcore_barrier()

        def iload(c, s):
            pltpu.async_copy(ei_h.at[tile, c], ring.at[s], isem[s])

        def iwait(c, s):
            pltpu.make_async_copy(ei_h.at[tile, c], ring.at[s],
                                  isem[s]).wait()

        def gstart(s, b):
            pltpu.async_copy(table_h.at[ring.at[s, 0]], rows[b], gsem[b])

        def gwait(s, b):
            pltpu.make_async_copy(table_h.at[ring.at[s, 0]], rows[b],
                                  gsem[b]).wait()

        def sstart(s, b):
            pltpu.async_copy(rows[b], acc_s.at[ring.at[s, 1]], ssem[b],
                             add=True)

        def swait(s, b):
            pltpu.make_async_copy(rows[b], acc_s.at[ring.at[s, 1]],
                                  ssem[b]).wait()

        def iter_ops(j, jm8, jm4, do_swait, do_iload, do_gather):
            gwait(jm8, jm4)
            sstart(jm8, jm4)
            if do_swait:
                swait((jm8 + 2) % NIDX, (jm4 + 2) % NBUF)
            if do_iload:
                iload(j + 6, (jm8 + 6) % NIDX)
            if do_gather:
                iwait(j + 2, (jm8 + 2) % NIDX)
                gstart((jm8 + 2) % NIDX, (jm4 + 2) % NBUF)

        for c in range(6):
            iload(c, c)
        for c in range(2):
            iwait(c, c)
            gstart(c, c)
        for j in (0, 1):
            iter_ops(j, j % NIDX, j % NBUF, False, True, True)
        for j in range(2, 10):
            iter_ops(j, j % NIDX, j % NBUF, True, True, True)

        def body(o, _):
            j0 = 10 + o * 8
            for t in range(8):
                iter_ops(j0 + t, (10 + t) % NIDX, (10 + t) % NBUF,
                         True, True, True)
            return 0
        lax.fori_loop(0, 13, body, 0)

        for j in range(114, 119):
            iter_ops(j, j % NIDX, j % NBUF, True, True, True)
        for j in range(119, 123):
            iter_ops(j, j % NIDX, j % NBUF, True, False, True)
        for j in (123, 124):
            jm8, jm4 = j % NIDX, j % NBUF
            gwait(jm8, jm4)
            sstart(jm8, jm4)
            swait((jm8 + 2) % NIDX, (jm4 + 2) % NBUF)
        for j in (123, 124):
            swait(j % NIDX, j % NBUF)
        plsc.subcore_barrier()

        pltpu.sync_copy(
            acc_s.at[pl.ds(sid * ROWS_PER_TILE, ROWS_PER_TILE)],
            out_h.at[pl.ds(cid * NPAD + sid * ROWS_PER_TILE, ROWS_PER_TILE)])

    return k(table, ei4)


def _mlp0(x, aggs, W1, b1, W2, b2):
    def body(x_r, a0_r, a1_r, w1_r, b1_r, w2_r, b2_r, o_r):
        h = x_r[...] + a0_r[...] + a1_r[...]
        h = jnp.maximum(
            jnp.dot(h, w1_r[...], preferred_element_type=jnp.float32)
            + b1_r[...], 0.0)
        h = jnp.dot(h, w2_r[...], preferred_element_type=jnp.float32) + b2_r[...]
        o_r[...] = jnp.maximum(h, 0.0)

    row = pl.BlockSpec((RB, DD), lambda i: (i, 0))
    row_hi = pl.BlockSpec((RB, DD), lambda i: (i + NRB, 0))
    full = pl.BlockSpec((DD, DD), lambda i: (0, 0))
    bias = pl.BlockSpec((1, DD), lambda i: (0, 0))
    return pl.pallas_call(
        body,
        grid=(NRB,),
        in_specs=[row, row, row_hi, full, bias, full, bias],
        out_specs=row,
        out_shape=jax.ShapeDtypeStruct((NPAD, DD), jnp.float32),
    )(x, aggs, aggs, W1, b1.reshape(1, DD), W2, b2.reshape(1, DD))


def _mlp1_pool(h0, aggs, W1, b1, W2, b2, batch3d):
    def body(h_r, a0_r, a1_r, w1_r, b1_r, w2_r, b2_r, bt_r, o_r, acc, cnt):
        i = pl.program_id(0)

        @pl.when(i == 0)
        def _():
            acc[...] = jnp.zeros_like(acc)
            cnt[...] = jnp.zeros_like(cnt)

        h = h_r[...] + a0_r[...] + a1_r[...]
        h = jnp.maximum(
            jnp.dot(h, w1_r[...], preferred_element_type=jnp.float32)
            + b1_r[...], 0.0)
        h = jnp.dot(h, w2_r[...], preferred_element_type=jnp.float32) + b2_r[...]

        seg = bt_r[...].reshape(1, RB)
        onehot = (jnp.broadcast_to(seg, (BB, RB))
                  == lax.broadcasted_iota(jnp.int32, (BB, RB), 0)
                  ).astype(jnp.float32)
        acc[...] += jnp.dot(onehot, h, preferred_element_type=jnp.float32)
        cnt[...] += jnp.broadcast_to(
            jnp.sum(onehot, axis=1, keepdims=True), (BB, DD))

        @pl.when(i == NRB - 1)
        def _():
            o_r[...] = acc[...] / jnp.maximum(cnt[...], 1.0)

    row = pl.BlockSpec((RB, DD), lambda i: (i, 0))
    row_hi = pl.BlockSpec((RB, DD), lambda i: (i + NRB, 0))
    full = pl.BlockSpec((DD, DD), lambda i: (0, 0))
    bias = pl.BlockSpec((1, DD), lambda i: (0, 0))
    return pl.pallas_call(
        body,
        grid=(NRB,),
        in_specs=[row, row, row_hi, full, bias, full, bias,
                  pl.BlockSpec((1, 1, RB), lambda i: (i, 0, 0))],
        out_specs=pl.BlockSpec((BB, DD), lambda i: (0, 0)),
        out_shape=jax.ShapeDtypeStruct((BB, DD), jnp.float32),
        scratch_shapes=[pltpu.VMEM((BB, DD), jnp.float32),
                        pltpu.VMEM((BB, DD), jnp.float32)],
    )(h0, aggs, aggs, W1, b1.reshape(1, DD), W2, b2.reshape(1, DD), batch3d)


def kernel(x, edge_index, batch, W1_0, b1_0, W2_0, b2_0, W1_1, b1_1, W2_1, b2_1):
    ei4 = edge_index.reshape(2, NTILES, NCHUNK, CH).transpose(1, 2, 0, 3)
    x_pad = jnp.pad(x, ((0, NPAD - NN), (0, 0)))
    batch3d = jnp.pad(batch, (0, NPAD - NN),
                      constant_values=BB).reshape(NRB, 1, RB)

    aggs = _sc_agg(x_pad, ei4)
    h0 = _mlp0(x_pad, aggs, W1_0, b1_0, W2_0, b2_0)
    aggs1 = _sc_agg(h0, ei4)
    return _mlp1_pool(h0, aggs1, W1_1, b1_1, W2_1, b2_1, batch3d)


# speedup vs baseline: 13.7275x; 1.0942x over previous
import functools

import jax
import jax.numpy as jnp
from jax import lax
from jax.experimental import pallas as pl
from jax.experimental.pallas import tpu as pltpu
from jax.experimental.pallas import tpu_sc as plsc

NN = 10000
NPAD = 10112
EE = 320000
DD = 128
BB = 64
NTILES = 32
E_PER_TILE = EE // NTILES
CH = 80
NCHUNK = E_PER_TILE // CH
NBUF = 4
NIDX = 8
ROWS_PER_TILE = NPAD // 16
ZCH = ROWS_PER_TILE // 8
RB = 632
NRB = NPAD // RB


def _sc_agg(table, ei4):
    mesh = plsc.VectorSubcoreMesh(core_axis_name="c", subcore_axis_name="s")

    @functools.partial(
        pl.kernel,
        out_type=jax.ShapeDtypeStruct((2 * NPAD, DD), jnp.float32),
        mesh=mesh,
        scratch_types=[
            pltpu.VMEM((NIDX, 2, CH), jnp.int32),
            [pltpu.VMEM((CH, DD), jnp.float32) for _ in range(NBUF)],
            pltpu.VMEM_SHARED((NPAD, DD), jnp.float32),
            [pltpu.SemaphoreType.DMA for _ in range(NIDX)],
            [pltpu.SemaphoreType.DMA for _ in range(NBUF)],
            [pltpu.SemaphoreType.DMA for _ in range(NBUF)],
        ],
    )
    def k(table_h, ei_h, out_h, ring, rows, acc_s, isem, gsem, ssem):
        cid = lax.axis_index("c")
        sid = lax.axis_index("s")
        tile = cid * 16 + sid

        def zrow(r, _):
            def zcol(j, _):
                rows[0][r, pl.ds(j * 16, 16)] = jnp.zeros((16,), jnp.float32)
                return 0
            return lax.fori_loop(0, DD // 16, zcol, 0)
        lax.fori_loop(0, CH, zrow, 0)

        def zacc(i, _):
            pltpu.sync_copy(
                rows[0].at[pl.ds(0, ZCH)],
                acc_s.at[pl.ds(sid * ROWS_PER_TILE + i * ZCH, ZCH)])
            return 0
        lax.fori_loop(0, ROWS_PER_TILE // ZCH, zacc, 0)
        plsc.subcore_barrier()

        def iload(c, s):
            pltpu.async_copy(ei_h.at[tile, c], ring.at[s], isem[s])

        def iwait(c, s):
            pltpu.make_async_copy(ei_h.at[tile, c], ring.at[s],
                                  isem[s]).wait()

        def gstart(s, b):
            pltpu.async_copy(table_h.at[ring.at[s, 0]], rows[b], gsem[b])

        def gwait(s, b):
            pltpu.make_async_copy(table_h.at[ring.at[s, 0]], rows[b],
                                  gsem[b]).wait()

        def sstart(s, b):
            pltpu.async_copy(rows[b], acc_s.at[ring.at[s, 1]], ssem[b],
                             add=True)

        def swait(s, b):
            pltpu.make_async_copy(rows[b], acc_s.at[ring.at[s, 1]],
                                  ssem[b]).wait()

        def iter_ops(j, jm8, jm4, do_swait, do_iload, do_gather):
            gwait(jm8, jm4)
            sstart(jm8, jm4)
            if do_swait:
                swait((jm8 + 7) % NIDX, (jm4 + 3) % NBUF)
            if do_iload:
                iload(j + 6, (jm8 + 6) % NIDX)
            if do_gather:
                iwait(j + 3, (jm8 + 3) % NIDX)
                gstart((jm8 + 3) % NIDX, (jm4 + 3) % NBUF)

        for c in range(6):
            iload(c, c)
        for c in range(3):
            iwait(c, c)
            gstart(c, c)
        iter_ops(0, 0, 0, False, True, True)
        for j in range(1, 10):
            iter_ops(j, j % NIDX, j % NBUF, True, True, True)

        def body(o, _):
            j0 = 10 + o * 8
            for t in range(8):
                iter_ops(j0 + t, (10 + t) % NIDX, (10 + t) % NBUF,
                         True, True, True)
            return 0
        lax.fori_loop(0, 13, body, 0)

        for j in range(114, 119):
            iter_ops(j, j % NIDX, j % NBUF, True, True, True)
        for j in range(119, 122):
            iter_ops(j, j % NIDX, j % NBUF, True, False, True)
        for j in (122, 123, 124):
            jm8, jm4 = j % NIDX, j % NBUF
            gwait(jm8, jm4)
            sstart(jm8, jm4)
            swait((jm8 + 7) % NIDX, (jm4 + 3) % NBUF)
        swait(124 % NIDX, 124 % NBUF)
        plsc.subcore_barrier()

        pltpu.sync_copy(
            acc_s.at[pl.ds(sid * ROWS_PER_TILE, ROWS_PER_TILE)],
            out_h.at[pl.ds(cid * NPAD + sid * ROWS_PER_TILE, ROWS_PER_TILE)])

    return k(table, ei4)


def _mlp0(x, aggs, W1, b1, W2, b2):
    def body(x_r, a0_r, a1_r, w1_r, b1_r, w2_r, b2_r, o_r):
        h = x_r[...] + a0_r[...] + a1_r[...]
        h = jnp.maximum(
            jnp.dot(h, w1_r[...], preferred_element_type=jnp.float32)
            + b1_r[...], 0.0)
        h = jnp.dot(h, w2_r[...], preferred_element_type=jnp.float32) + b2_r[...]
        o_r[...] = jnp.maximum(h, 0.0)

    row = pl.BlockSpec((RB, DD), lambda i: (i, 0))
    row_hi = pl.BlockSpec((RB, DD), lambda i: (i + NRB, 0))
    full = pl.BlockSpec((DD, DD), lambda i: (0, 0))
    bias = pl.BlockSpec((1, DD), lambda i: (0, 0))
    return pl.pallas_call(
        body,
        grid=(NRB,),
        in_specs=[row, row, row_hi, full, bias, full, bias],
        out_specs=row,
        out_shape=jax.ShapeDtypeStruct((NPAD, DD), jnp.float32),
    )(x, aggs, aggs, W1, b1.reshape(1, DD), W2, b2.reshape(1, DD))


def _mlp1_pool(h0, aggs, W1, b1, W2, b2, batch3d):
    def body(h_r, a0_r, a1_r, w1_r, b1_r, w2_r, b2_r, bt_r, o_r, acc, cnt):
        i = pl.program_id(0)

        @pl.when(i == 0)
        def _():
            acc[...] = jnp.zeros_like(acc)
            cnt[...] = jnp.zeros_like(cnt)

        h = h_r[...] + a0_r[...] + a1_r[...]
        h = jnp.maximum(
            jnp.dot(h, w1_r[...], preferred_element_type=jnp.float32)
            + b1_r[...], 0.0)
        h = jnp.dot(h, w2_r[...], preferred_element_type=jnp.float32) + b2_r[...]

        seg = bt_r[...].reshape(1, RB)
        onehot = (jnp.broadcast_to(seg, (BB, RB))
                  == lax.broadcasted_iota(jnp.int32, (BB, RB), 0)
                  ).astype(jnp.float32)
        acc[...] += jnp.dot(onehot, h, preferred_element_type=jnp.float32)
        cnt[...] += jnp.broadcast_to(
            jnp.sum(onehot, axis=1, keepdims=True), (BB, DD))

        @pl.when(i == NRB - 1)
        def _():
            o_r[...] = acc[...] / jnp.maximum(cnt[...], 1.0)

    row = pl.BlockSpec((RB, DD), lambda i: (i, 0))
    row_hi = pl.BlockSpec((RB, DD), lambda i: (i + NRB, 0))
    full = pl.BlockSpec((DD, DD), lambda i: (0, 0))
    bias = pl.BlockSpec((1, DD), lambda i: (0, 0))
    return pl.pallas_call(
        body,
        grid=(NRB,),
        in_specs=[row, row, row_hi, full, bias, full, bias,
                  pl.BlockSpec((1, 1, RB), lambda i: (i, 0, 0))],
        out_specs=pl.BlockSpec((BB, DD), lambda i: (0, 0)),
        out_shape=jax.ShapeDtypeStruct((BB, DD), jnp.float32),
        scratch_shapes=[pltpu.VMEM((BB, DD), jnp.float32),
                        pltpu.VMEM((BB, DD), jnp.float32)],
    )(h0, aggs, aggs, W1, b1.reshape(1, DD), W2, b2.reshape(1, DD), batch3d)


def kernel(x, edge_index, batch, W1_0, b1_0, W2_0, b2_0, W1_1, b1_1, W2_1, b2_1):
    ei4 = edge_index.reshape(2, NTILES, NCHUNK, CH).transpose(1, 2, 0, 3)
    x_pad = jnp.pad(x, ((0, NPAD - NN), (0, 0)))
    batch3d = jnp.pad(batch, (0, NPAD - NN),
                      constant_values=BB).reshape(NRB, 1, RB)

    aggs = _sc_agg(x_pad, ei4)
    h0 = _mlp0(x_pad, aggs, W1_0, b1_0, W2_0, b2_0)
    aggs1 = _sc_agg(h0, ei4)
    return _mlp1_pool(h0, aggs1, W1_1, b1_1, W2_1, b2_1, batch3d)
